# baseline (speedup 1.0000x reference)
import functools

import jax
import jax.numpy as jnp
from jax import lax
from jax.experimental import pallas as pl
from jax.experimental.pallas import tpu as pltpu
from jax.experimental.pallas import tpu_sc as plsc

L = 16


@functools.cache
def _build(B, T, S, V):
  NC, NS = 2, 16
  NW = NC * NS
  rows = B * T
  assert rows % NW == 0
  rpw = rows // NW
  assert (T % rpw == 0) and (S % L == 0) and (V % L == 0) and rpw % 2 == 0
  ngrp = S // L

  mesh = plsc.VectorSubcoreMesh(core_axis_name="c", subcore_axis_name="s")

  @functools.partial(
      pl.kernel,
      out_type=jax.ShapeDtypeStruct((B, T, V), jnp.float32),
      mesh=mesh,
      scratch_types=[
          pltpu.VMEM((V,), jnp.float32),
          pltpu.VMEM((V,), jnp.float32),
          pltpu.VMEM((S,), jnp.float32),
          pltpu.VMEM((S,), jnp.float32),
          pltpu.VMEM((S,), jnp.int32),
          pltpu.SemaphoreType.DMA,
          pltpu.SemaphoreType.DMA,
          pltpu.SemaphoreType.DMA,
      ],
      compiler_params=pltpu.CompilerParams(
          needs_layout_passes=False,
          use_tc_tiling_on_sc=True,
          skip_device_barrier=True,
      ),
  )
  def scatter_rows(p_hbm, ids_hbm, out_hbm,
                   buf0, buf1, prow0, prow1, ids_v, sem0, sem1, semp):
    wid = lax.axis_index("s") * NC + lax.axis_index("c")
    base = wid * rpw
    batch = base // T
    t0 = base % T

    pltpu.sync_copy(ids_hbm.at[batch], ids_v)
    cpp = pltpu.async_copy(p_hbm.at[batch, t0], prow0, semp)

    zero = jnp.zeros((L,), jnp.float32)

    def zero0(i, _):
      buf0[pl.ds(i * 4 * L, L)] = zero
      buf0[pl.ds((i * 4 + 1) * L, L)] = zero
      buf0[pl.ds((i * 4 + 2) * L, L)] = zero
      buf0[pl.ds((i * 4 + 3) * L, L)] = zero
      return 0

    def zero1(i, _):
      buf1[pl.ds(i * 4 * L, L)] = zero
      buf1[pl.ds((i * 4 + 1) * L, L)] = zero
      buf1[pl.ds((i * 4 + 2) * L, L)] = zero
      buf1[pl.ds((i * 4 + 3) * L, L)] = zero
      return 0

    lax.fori_loop(0, V // (4 * L), zero0, 0)

    ivs = [ids_v[pl.ds(g * L, L)] for g in range(ngrp)]

    def add_groups(buf, prow):
      for g in range(ngrp):
        plsc.addupdate_scatter(buf, [ivs[g]], prow[pl.ds(g * L, L)])

    def scatter_row(buf, prow, t):
      pltpu.sync_copy(p_hbm.at[batch, t], prow)
      add_groups(buf, prow)

    def unscatter(buf):
      for g in range(ngrp):
        plsc.store_scatter(buf, [ivs[g]], zero)

    cpp.wait()
    add_groups(buf0, prow0)
    cp0 = pltpu.async_copy(buf0, out_hbm.at[batch, t0], sem0)
    lax.fori_loop(0, V // (4 * L), zero1, 0)
    scatter_row(buf1, prow1, t0 + 1)
    cp1 = pltpu.async_copy(buf1, out_hbm.at[batch, t0 + 1], sem1)

    def row_body(q, _):
      t = t0 + 2 * q
      pltpu.make_async_copy(buf0, out_hbm.at[batch, t], sem0).wait()
      unscatter(buf0)
      scatter_row(buf0, prow0, t)
      pltpu.async_copy(buf0, out_hbm.at[batch, t], sem0)
      pltpu.make_async_copy(buf1, out_hbm.at[batch, t + 1], sem1).wait()
      unscatter(buf1)
      scatter_row(buf1, prow1, t + 1)
      pltpu.async_copy(buf1, out_hbm.at[batch, t + 1], sem1)
      return 0

    lax.fori_loop(1, rpw // 2, row_body, 0)

    cpl0 = pltpu.make_async_copy(buf0, out_hbm.at[batch, t0], sem0)
    cpl0.wait()
    cpl1 = pltpu.make_async_copy(buf1, out_hbm.at[batch, t0], sem1)
    cpl1.wait()
    del cp0, cp1

  return scatter_rows


def kernel(p_source_position, p_target_vocab, input_source):
  B, T, S = p_source_position.shape
  V = p_target_vocab.shape[-1]
  fn = _build(B, T, S, V)
  return fn(p_source_position.astype(jnp.float32),
            input_source.astype(jnp.int32))


# speedup vs baseline: 1.0036x; 1.0036x over previous
import functools

import jax
import jax.numpy as jnp
from jax import lax
from jax.experimental import pallas as pl
from jax.experimental.pallas import tpu as pltpu
from jax.experimental.pallas import tpu_sc as plsc

L = 16


@functools.cache
def _build(B, T, S, V):
  NC, NS = 2, 16
  NW = NC * NS
  rows = B * T
  assert rows % NW == 0
  rpw = rows // NW
  assert (T % rpw == 0) and (S % L == 0) and (V % L == 0) and rpw % 2 == 0
  ngrp = S // L

  mesh = plsc.VectorSubcoreMesh(core_axis_name="c", subcore_axis_name="s")

  @functools.partial(
      pl.kernel,
      out_type=jax.ShapeDtypeStruct((B, T, V), jnp.float32),
      mesh=mesh,
      scratch_types=[
          pltpu.VMEM((V,), jnp.float32),
          pltpu.VMEM((V,), jnp.float32),
          pltpu.VMEM((S,), jnp.float32),
          pltpu.VMEM((S,), jnp.float32),
          pltpu.VMEM((S,), jnp.int32),
          pltpu.SemaphoreType.DMA,
          pltpu.SemaphoreType.DMA,
          pltpu.SemaphoreType.DMA,
      ],
      compiler_params=pltpu.CompilerParams(
          needs_layout_passes=False,
          use_tc_tiling_on_sc=True,
      ),
  )
  def scatter_rows(p_hbm, ids_hbm, out_hbm,
                   buf0, buf1, prow0, prow1, ids_v, sem0, sem1, semp):
    wid = lax.axis_index("s") * NC + lax.axis_index("c")
    base = wid * rpw
    batch = base // T
    t0 = base % T

    pltpu.sync_copy(ids_hbm.at[batch], ids_v)
    cpp = pltpu.async_copy(p_hbm.at[batch, t0], prow0, semp)

    zero = jnp.zeros((L,), jnp.float32)

    def zero0(i, _):
      buf0[pl.ds(i * 4 * L, L)] = zero
      buf0[pl.ds((i * 4 + 1) * L, L)] = zero
      buf0[pl.ds((i * 4 + 2) * L, L)] = zero
      buf0[pl.ds((i * 4 + 3) * L, L)] = zero
      return 0

    def zero1(i, _):
      buf1[pl.ds(i * 4 * L, L)] = zero
      buf1[pl.ds((i * 4 + 1) * L, L)] = zero
      buf1[pl.ds((i * 4 + 2) * L, L)] = zero
      buf1[pl.ds((i * 4 + 3) * L, L)] = zero
      return 0

    lax.fori_loop(0, V // (4 * L), zero0, 0)

    ivs = [ids_v[pl.ds(g * L, L)] for g in range(ngrp)]

    def add_groups(buf, prow):
      for g in range(ngrp):
        plsc.addupdate_scatter(buf, [ivs[g]], prow[pl.ds(g * L, L)])

    def scatter_row(buf, prow, t):
      pltpu.sync_copy(p_hbm.at[batch, t], prow)
      add_groups(buf, prow)

    def unscatter(buf):
      for g in range(ngrp):
        plsc.store_scatter(buf, [ivs[g]], zero)

    cpp.wait()
    add_groups(buf0, prow0)
    cp0 = pltpu.async_copy(buf0, out_hbm.at[batch, t0], sem0)
    lax.fori_loop(0, V // (4 * L), zero1, 0)
    scatter_row(buf1, prow1, t0 + 1)
    cp1 = pltpu.async_copy(buf1, out_hbm.at[batch, t0 + 1], sem1)

    def row_body(q, _):
      t = t0 + 2 * q
      pltpu.make_async_copy(buf0, out_hbm.at[batch, t], sem0).wait()
      unscatter(buf0)
      scatter_row(buf0, prow0, t)
      pltpu.async_copy(buf0, out_hbm.at[batch, t], sem0)
      pltpu.make_async_copy(buf1, out_hbm.at[batch, t + 1], sem1).wait()
      unscatter(buf1)
      scatter_row(buf1, prow1, t + 1)
      pltpu.async_copy(buf1, out_hbm.at[batch, t + 1], sem1)
      return 0

    lax.fori_loop(1, rpw // 2, row_body, 0)

    pltpu.make_async_copy(buf0, out_hbm.at[batch, t0], sem0).wait()
    pltpu.make_async_copy(buf1, out_hbm.at[batch, t0], sem1).wait()
    del cp0, cp1

  return scatter_rows


def kernel(p_source_position, p_target_vocab, input_source):
  B, T, S = p_source_position.shape
  V = p_target_vocab.shape[-1]
  fn = _build(B, T, S, V)
  return fn(p_source_position.astype(jnp.float32),
            input_source.astype(jnp.int32))
